# Initial kernel scaffold; baseline (speedup 1.0000x reference)
import jax
import jax.numpy as jnp
from jax.experimental import pallas as pl


def kernel(x, Wq, Wk, Wv, Wproj, alpha_raw, beta_raw, state):
    raise NotImplementedError("write your pallas kernel here")


# speedup vs baseline: 4.7887x; 4.7887x over previous
import functools

import jax
import jax.numpy as jnp
from jax import lax
from jax.experimental import pallas as pl
from jax.experimental.pallas import tpu as pltpu


def _titans_body(H, D, L, NC, x_ref, wq_ref, wk_ref, wv_ref, wp_ref,
                 ab_ref, state_ref, out_ref, stateout_ref,
                 s_ref, qs_ref, ks_ref, vs_ref, ys_ref):
    c = pl.program_id(1)
    f32 = jnp.float32
    dn = (((1,), (1,)), ((), ()))

    @pl.when(c == 0)
    def _():
        s_ref[...] = state_ref[0]

    xb = x_ref[0]
    qs_ref[...] = lax.dot_general(xb, wq_ref[...], dn, preferred_element_type=f32)
    ks_ref[...] = lax.dot_general(xb, wk_ref[...], dn, preferred_element_type=f32)
    vs_ref[...] = lax.dot_general(xb, wv_ref[...], dn, preferred_element_type=f32)

    ab = 0.5 * jax.nn.sigmoid(ab_ref[...])

    row = lax.broadcasted_iota(jnp.int32, (L, L), 0)
    col = lax.broadcasted_iota(jnp.int32, (L, L), 1)
    smask = col < row

    for h in range(H):
        sl = slice(h * D, (h + 1) * D)
        Q = qs_ref[:, sl]
        K = ks_ref[:, sl]
        V = vs_ref[:, sl]
        nrm = jnp.sqrt(jnp.sum(K * K, axis=1, keepdims=True))
        K = K / jnp.maximum(nrm, 1e-12)
        alpha = ab[0:1, h:h + 1]
        beta = ab[1:2, h:h + 1]
        S = s_ref[h]

        A = lax.dot_general(K, K, dn, preferred_element_type=f32)
        Nm = jnp.where(smask, alpha * A, 0.0)
        R = beta * V - alpha * lax.dot_general(K, S, dn, preferred_element_type=f32)

        powers = []
        P = Nm
        p = 2
        while p <= L // 2:
            P = jnp.dot(P, P, preferred_element_type=f32)
            powers.append(P)
            p *= 2
        U = R
        for P in reversed(powers):
            U = U + jnp.dot(P, U, preferred_element_type=f32)
        U = U - jnp.dot(Nm, U, preferred_element_type=f32)

        QK = lax.dot_general(Q, K, dn, preferred_element_type=f32)
        Y = lax.dot_general(Q, S, dn, preferred_element_type=f32) + \
            jnp.dot(jnp.where(smask, QK, 0.0), U, preferred_element_type=f32)
        s_ref[h] = S + lax.dot_general(
            U, K, (((0,), (0,)), ((), ())), preferred_element_type=f32)
        ys_ref[:, sl] = Y

    out_ref[0] = lax.dot_general(ys_ref[...], wp_ref[...], dn,
                                 preferred_element_type=f32)

    @pl.when(c == NC - 1)
    def _():
        stateout_ref[0] = s_ref[...]


def kernel(x, Wq, Wk, Wv, Wproj, alpha_raw, beta_raw, state):
    B, T, C = x.shape
    H = alpha_raw.shape[1]
    D = C // H
    L = 128 if T % 128 == 0 else T
    NC = T // L
    ab = jnp.stack([alpha_raw.reshape(H), beta_raw.reshape(H)])

    body = functools.partial(_titans_body, H, D, L, NC)
    wspec = pl.BlockSpec((C, C), lambda b, c: (0, 0))
    out, state_f = pl.pallas_call(
        body,
        grid=(B, NC),
        in_specs=[
            pl.BlockSpec((1, L, C), lambda b, c: (b, c, 0)),
            wspec, wspec, wspec, wspec,
            pl.BlockSpec((2, H), lambda b, c: (0, 0)),
            pl.BlockSpec((1, H, D, D), lambda b, c: (b, 0, 0, 0)),
        ],
        out_specs=[
            pl.BlockSpec((1, L, C), lambda b, c: (b, c, 0)),
            pl.BlockSpec((1, H, D, D), lambda b, c: (b, 0, 0, 0)),
        ],
        out_shape=[
            jax.ShapeDtypeStruct((B, T, C), jnp.float32),
            jax.ShapeDtypeStruct((B, H, D, D), jnp.float32),
        ],
        scratch_shapes=[
            pltpu.VMEM((H, D, D), jnp.float32),
            pltpu.VMEM((L, C), jnp.float32),
            pltpu.VMEM((L, C), jnp.float32),
            pltpu.VMEM((L, C), jnp.float32),
            pltpu.VMEM((L, C), jnp.float32),
        ],
        compiler_params=pltpu.CompilerParams(
            dimension_semantics=("parallel", "arbitrary"),
            vmem_limit_bytes=56 * 1024 * 1024,
        ),
        name="titans_l2_chunked",
    )(x, Wq, Wk, Wv, Wproj, ab, state)
    return out, state_f


# speedup vs baseline: 18.9041x; 3.9477x over previous
import functools

import jax
import jax.numpy as jnp
from jax import lax
from jax.experimental import pallas as pl
from jax.experimental.pallas import tpu as pltpu


def _titans_body(H, D, L, NC, x_ref, wq_ref, wk_ref, wv_ref, wp_ref,
                 ab_ref, state_ref, out_ref, stateout_ref,
                 s_ref, qt_ref, kt_ref, vt_ref, yt_ref):
    c = pl.program_id(1)
    f32 = jnp.float32
    c11 = (((1,), (1,)), ((), ()))
    c00 = (((0,), (0,)), ((), ()))

    @pl.when(c == 0)
    def _():
        s_ref[...] = state_ref[0]

    xb = x_ref[0]
    qt_ref[...] = lax.dot_general(wq_ref[...], xb, c11, preferred_element_type=f32)
    kt_ref[...] = lax.dot_general(wk_ref[...], xb, c11, preferred_element_type=f32)
    vt_ref[...] = lax.dot_general(wv_ref[...], xb, c11, preferred_element_type=f32)

    ab = 0.5 * jax.nn.sigmoid(ab_ref[...])

    row = lax.broadcasted_iota(jnp.int32, (L, L), 0)
    col = lax.broadcasted_iota(jnp.int32, (L, L), 1)
    umask = row < col

    GRP = 4
    for h0 in range(0, H, GRP):
        hs = range(h0, min(h0 + GRP, H))
        Qt, Kt, Vt, S, al, be = {}, {}, {}, {}, {}, {}
        for h in hs:
            sl = slice(h * D, (h + 1) * D)
            Qt[h] = qt_ref[sl, :]
            Vt[h] = vt_ref[sl, :]
            K = kt_ref[sl, :]
            nrm = jnp.sqrt(jnp.sum(K * K, axis=0, keepdims=True))
            Kt[h] = K / jnp.maximum(nrm, 1e-12)
            al[h] = ab[0:1, h:h + 1]
            be[h] = ab[1:2, h:h + 1]
            S[h] = s_ref[h]
        Nt, Ut = {}, {}
        for h in hs:
            At = lax.dot_general(Kt[h], Kt[h], c00, preferred_element_type=f32)
            Nt[h] = jnp.where(umask, al[h] * At, 0.0)
        for h in hs:
            Rt = be[h] * Vt[h] - al[h] * jnp.dot(S[h], Kt[h],
                                                 preferred_element_type=f32)
            Ut[h] = Rt - jnp.dot(Rt, Nt[h], preferred_element_type=f32)
        P = dict(Nt)
        p = 2
        while p <= L // 2:
            for h in hs:
                P[h] = jnp.dot(P[h], P[h], preferred_element_type=f32)
            for h in hs:
                Ut[h] = Ut[h] + jnp.dot(Ut[h], P[h], preferred_element_type=f32)
            p *= 2
        for h in hs:
            G = lax.dot_general(Kt[h], Qt[h], c00, preferred_element_type=f32)
            Yt = jnp.dot(S[h], Qt[h], preferred_element_type=f32) + \
                jnp.dot(Ut[h], jnp.where(umask, G, 0.0),
                        preferred_element_type=f32)
            yt_ref[h * D:(h + 1) * D, :] = Yt
        for h in hs:
            s_ref[h] = S[h] + lax.dot_general(Ut[h], Kt[h], c11,
                                              preferred_element_type=f32)

    out_ref[0] = lax.dot_general(yt_ref[...], wp_ref[...],
                                 (((0,), (1,)), ((), ())),
                                 preferred_element_type=f32)

    @pl.when(c == NC - 1)
    def _():
        stateout_ref[0] = s_ref[...]


def kernel(x, Wq, Wk, Wv, Wproj, alpha_raw, beta_raw, state):
    B, T, C = x.shape
    H = alpha_raw.shape[1]
    D = C // H
    L = 128 if T % 128 == 0 else T
    NC = T // L
    ab = jnp.stack([alpha_raw.reshape(H), beta_raw.reshape(H)])

    body = functools.partial(_titans_body, H, D, L, NC)
    wspec = pl.BlockSpec((C, C), lambda b, c: (0, 0))
    out, state_f = pl.pallas_call(
        body,
        grid=(B, NC),
        in_specs=[
            pl.BlockSpec((1, L, C), lambda b, c: (b, c, 0)),
            wspec, wspec, wspec, wspec,
            pl.BlockSpec((2, H), lambda b, c: (0, 0)),
            pl.BlockSpec((1, H, D, D), lambda b, c: (b, 0, 0, 0)),
        ],
        out_specs=[
            pl.BlockSpec((1, L, C), lambda b, c: (b, c, 0)),
            pl.BlockSpec((1, H, D, D), lambda b, c: (b, 0, 0, 0)),
        ],
        out_shape=[
            jax.ShapeDtypeStruct((B, T, C), jnp.float32),
            jax.ShapeDtypeStruct((B, H, D, D), jnp.float32),
        ],
        scratch_shapes=[
            pltpu.VMEM((H, D, D), jnp.float32),
            pltpu.VMEM((C, L), jnp.float32),
            pltpu.VMEM((C, L), jnp.float32),
            pltpu.VMEM((C, L), jnp.float32),
            pltpu.VMEM((C, L), jnp.float32),
        ],
        compiler_params=pltpu.CompilerParams(
            dimension_semantics=("parallel", "arbitrary"),
            vmem_limit_bytes=56 * 1024 * 1024,
        ),
        name="titans_l2_chunked",
    )(x, Wq, Wk, Wv, Wproj, ab, state)
    return out, state_f


# speedup vs baseline: 27.4373x; 1.4514x over previous
import functools

import jax
import jax.numpy as jnp
from jax import lax
from jax.experimental import pallas as pl
from jax.experimental.pallas import tpu as pltpu


def _titans_body(H, D, L, NC, x_ref, wq_ref, wk_ref, wv_ref, wp_ref,
                 ab_ref, state_ref, out_ref, stateout_ref,
                 s_ref, qt_ref, kt_ref, vt_ref, yt_ref):
    c = pl.program_id(1)
    f32 = jnp.float32
    c11 = (((1,), (1,)), ((), ()))
    c00 = (((0,), (0,)), ((), ()))

    @pl.when(c == 0)
    def _():
        s_ref[...] = state_ref[0]

    xb = x_ref[0]
    qt_ref[...] = lax.dot_general(wq_ref[...], xb, c11, preferred_element_type=f32)
    kt_ref[...] = lax.dot_general(wk_ref[...], xb, c11, preferred_element_type=f32)
    vt_ref[...] = lax.dot_general(wv_ref[...], xb, c11, preferred_element_type=f32)

    ab = 0.5 * jax.nn.sigmoid(ab_ref[...])

    row = lax.broadcasted_iota(jnp.int32, (L, L), 0)
    col = lax.broadcasted_iota(jnp.int32, (L, L), 1)
    umask = row < col

    GRP = 16
    for h0 in range(0, H, GRP):
        hs = range(h0, min(h0 + GRP, H))
        Qt, Kt, Vt, S, al, be = {}, {}, {}, {}, {}, {}
        for h in hs:
            sl = slice(h * D, (h + 1) * D)
            Qt[h] = qt_ref[sl, :]
            Vt[h] = vt_ref[sl, :]
            K = kt_ref[sl, :]
            nrm = jnp.sqrt(jnp.sum(K * K, axis=0, keepdims=True))
            Kt[h] = K / jnp.maximum(nrm, 1e-12)
            al[h] = ab[0:1, h:h + 1]
            be[h] = ab[1:2, h:h + 1]
            S[h] = s_ref[h]
        Nt, Ut = {}, {}
        for h in hs:
            At = lax.dot_general(Kt[h], Kt[h], c00, preferred_element_type=f32)
            Nt[h] = jnp.where(umask, al[h] * At, 0.0)
        for h in hs:
            Rt = be[h] * Vt[h] - al[h] * jnp.dot(S[h], Kt[h],
                                                 preferred_element_type=f32)
            Ut[h] = Rt - jnp.dot(Rt, Nt[h], preferred_element_type=f32)
        P = dict(Nt)
        p = 2
        while p <= L // 2:
            for h in hs:
                P[h] = jnp.dot(P[h], P[h], preferred_element_type=f32)
            for h in hs:
                Ut[h] = Ut[h] + jnp.dot(Ut[h], P[h], preferred_element_type=f32)
            p *= 2
        for h in hs:
            G = lax.dot_general(Kt[h], Qt[h], c00, preferred_element_type=f32)
            Yt = jnp.dot(S[h], Qt[h], preferred_element_type=f32) + \
                jnp.dot(Ut[h], jnp.where(umask, G, 0.0),
                        preferred_element_type=f32)
            yt_ref[h * D:(h + 1) * D, :] = Yt
        for h in hs:
            s_ref[h] = S[h] + lax.dot_general(Ut[h], Kt[h], c11,
                                              preferred_element_type=f32)

    out_ref[0] = lax.dot_general(yt_ref[...], wp_ref[...],
                                 (((0,), (1,)), ((), ())),
                                 preferred_element_type=f32)

    @pl.when(c == NC - 1)
    def _():
        stateout_ref[0] = s_ref[...]


def kernel(x, Wq, Wk, Wv, Wproj, alpha_raw, beta_raw, state):
    B, T, C = x.shape
    H = alpha_raw.shape[1]
    D = C // H
    L = 256 if T % 256 == 0 else (128 if T % 128 == 0 else T)
    NC = T // L
    ab = jnp.stack([alpha_raw.reshape(H), beta_raw.reshape(H)])

    body = functools.partial(_titans_body, H, D, L, NC)
    wspec = pl.BlockSpec((C, C), lambda b, c: (0, 0))
    out, state_f = pl.pallas_call(
        body,
        grid=(B, NC),
        in_specs=[
            pl.BlockSpec((1, L, C), lambda b, c: (b, c, 0)),
            wspec, wspec, wspec, wspec,
            pl.BlockSpec((2, H), lambda b, c: (0, 0)),
            pl.BlockSpec((1, H, D, D), lambda b, c: (b, 0, 0, 0)),
        ],
        out_specs=[
            pl.BlockSpec((1, L, C), lambda b, c: (b, c, 0)),
            pl.BlockSpec((1, H, D, D), lambda b, c: (b, 0, 0, 0)),
        ],
        out_shape=[
            jax.ShapeDtypeStruct((B, T, C), jnp.float32),
            jax.ShapeDtypeStruct((B, H, D, D), jnp.float32),
        ],
        scratch_shapes=[
            pltpu.VMEM((H, D, D), jnp.float32),
            pltpu.VMEM((C, L), jnp.float32),
            pltpu.VMEM((C, L), jnp.float32),
            pltpu.VMEM((C, L), jnp.float32),
            pltpu.VMEM((C, L), jnp.float32),
        ],
        compiler_params=pltpu.CompilerParams(
            dimension_semantics=("parallel", "arbitrary"),
            vmem_limit_bytes=56 * 1024 * 1024,
        ),
        name="titans_l2_chunked",
    )(x, Wq, Wk, Wv, Wproj, ab, state)
    return out, state_f
